# Initial kernel scaffold; baseline (speedup 1.0000x reference)
import jax
import jax.numpy as jnp
from jax.experimental import pallas as pl


def kernel(src_tab, seg_tab, w, q_W1, q_b1, q_W2, q_b2, c1_w, c1_root, c1_b, c2_w, c2_root, c2_b, lin_W, lin_b, src, seg, edge_index, edge_type):
    raise NotImplementedError("write your pallas kernel here")


# speedup vs baseline: 3.1739x; 3.1739x over previous
import functools

import jax
import jax.numpy as jnp
from jax import lax
from jax.experimental import pallas as pl
from jax.experimental.pallas import tpu as pltpu
from jax.experimental.pallas import tpu_sc as plsc

N = 10000
E = 320000
L = 128
EMB = 128
NREL = 4
TXT = 256

NC = 2
NS = 16
NW = NC * NS

NPW = (N + NW - 1) // NW
EB = 128
EROWS = 2560
EPAD = EROWS * EB
RPT = EROWS // NS
SLABS = RPT // 16

DHALF = 5000
EACC = 5248
EDUMMY = 5120
ETR = EACC // NS

CR = 1024
CACC = 4224
CDUMMY = 4096
CTR = CACC // NS
NPD = 10 * CR

_MESH = plsc.VectorSubcoreMesh(core_axis_name="c", subcore_axis_name="s")


def _nrsqrt(t):
    y = lax.bitcast_convert_type(t, jnp.int32)
    y = jnp.int32(0x5F3759DF) - (y >> 1)
    y = lax.bitcast_convert_type(y, jnp.float32)
    for _ in range(3):
        y = y * (1.5 - 0.5 * t * y * y)
    return y


@functools.partial(
    pl.kernel,
    mesh=_MESH,
    out_type=jax.ShapeDtypeStruct((N, L, 16), jnp.float32),
    scratch_types=[
        pltpu.VMEM((L,), jnp.int32),
        pltpu.VMEM((L, EMB), jnp.float32),
        pltpu.VMEM((L, 16), jnp.float32),
        pltpu.VMEM((EMB,), jnp.float32),
        pltpu.SemaphoreType.DMA,
    ],
)
def _sc_embed(tab_hbm, src_hbm, w_hbm, out_hbm, idx_v, S_v, P_v, w_v, sem):
    core = lax.axis_index("c")
    sub = lax.axis_index("s")
    wid = core * NS + sub
    base = wid * NPW
    num = jnp.maximum(0, jnp.minimum(NPW, N - base))

    pltpu.sync_copy(w_hbm, w_v)

    def node_body(i, carry):
        node = base + i
        pltpu.sync_copy(src_hbm.at[pl.ds(node * L, L)], idx_v)
        pltpu.async_copy(tab_hbm.at[idx_v], S_v, sem).wait()

        def ss_row(r, accs):
            return tuple(
                accs[j] + S_v[r, pl.ds(j * 16, 16)] * S_v[r, pl.ds(j * 16, 16)]
                for j in range(8)
            )
        accs = lax.fori_loop(0, L, ss_row,
                             tuple(jnp.zeros((16,), jnp.float32) for _ in range(8)))

        cs = []
        for j in range(8):
            wv = w_v[pl.ds(j * 16, 16)]
            t = wv * wv * accs[j]
            norm = t * _nrsqrt(t)
            cs.append(wv / jnp.maximum(norm, 1e-12))

        def p_row(r, carry2):
            p = S_v[r, pl.ds(0, 16)] * cs[0]
            for j in range(1, 8):
                p = p + S_v[r, pl.ds(j * 16, 16)] * cs[j]
            P_v[r] = p
            return carry2
        lax.fori_loop(0, L, p_row, 0)

        pltpu.sync_copy(P_v, out_hbm.at[node])
        return carry

    lax.fori_loop(0, num, node_body, 0)


@functools.partial(
    pl.kernel,
    mesh=_MESH,
    out_type=jax.ShapeDtypeStruct((10, 4 * CR, EMB), jnp.float32),
    scratch_types=[
        pltpu.VMEM((CTR, EMB), jnp.float32),
        pltpu.VMEM((16, EB), jnp.int32),
        pltpu.VMEM((EB, EMB), jnp.float32),
        pltpu.VMEM_SHARED((CACC, EMB), jnp.float32),
    ],
)
def _sc_count(scidx_hbm, cnt_hbm, Z_v, si_v, ones_v, acc_sh):
    core = lax.axis_index("c")
    sub = lax.axis_index("s")

    z16 = jnp.zeros((16,), jnp.float32)
    o16 = jnp.ones((16,), jnp.float32)

    def zb(r, c):
        for j in range(EMB // 16):
            Z_v[r, pl.ds(j * 16, 16)] = z16
        return c
    lax.fori_loop(0, CTR, zb, 0)

    def ob(r, c):
        for j in range(EMB // 16):
            ones_v[r, pl.ds(j * 16, 16)] = o16
        return c
    lax.fori_loop(0, EB, ob, 0)

    for k in range(5):
        rg = core * 5 + k
        pltpu.sync_copy(Z_v, acc_sh.at[pl.ds(sub * CTR, CTR)])
        plsc.subcore_barrier()

        def slab(s, c):
            pltpu.sync_copy(scidx_hbm.at[rg, pl.ds(sub * RPT + s * 16, 16)], si_v)

            def batch(b, c2):
                pltpu.sync_copy(ones_v, acc_sh.at[si_v.at[b]], add=True)
                return c2
            lax.fori_loop(0, 16, batch, 0)
            return c
        lax.fori_loop(0, SLABS, slab, 0)

        plsc.subcore_barrier()
        pltpu.sync_copy(acc_sh.at[pl.ds(sub * (4 * CR // NS), 4 * CR // NS)],
                        cnt_hbm.at[rg, pl.ds(sub * (4 * CR // NS), 4 * CR // NS)])
        plsc.subcore_barrier()


@functools.partial(
    pl.kernel,
    mesh=_MESH,
    out_type=jax.ShapeDtypeStruct((NC, DHALF, EMB), jnp.float32),
    scratch_types=[
        pltpu.VMEM((ETR, EMB), jnp.float32),
        pltpu.VMEM((16, EB), jnp.int32),
        pltpu.VMEM((16, EB), jnp.int32),
        pltpu.VMEM((16, EB), jnp.int32),
        pltpu.VMEM((EB, EMB), jnp.float32),
        pltpu.VMEM((EB, EMB), jnp.float32),
        pltpu.VMEM_SHARED((EACC, EMB), jnp.float32),
        pltpu.SemaphoreType.DMA,
    ],
)
def _sc_edge(y_hbm, ic_hbm, vg_hbm, sg_hbm, se_hbm, agg_hbm,
             Z_v, vg_v, sg_v, se_v, rows_v, scale_v, acc_sh, sem):
    core = lax.axis_index("c")
    sub = lax.axis_index("s")

    z16 = jnp.zeros((16,), jnp.float32)

    def zb(r, c):
        for j in range(EMB // 16):
            Z_v[r, pl.ds(j * 16, 16)] = z16
        return c
    lax.fori_loop(0, ETR, zb, 0)

    pltpu.sync_copy(Z_v, acc_sh.at[pl.ds(sub * ETR, ETR)])
    plsc.subcore_barrier()

    def slab(s, c):
        pltpu.sync_copy(vg_hbm.at[pl.ds(sub * RPT + s * 16, 16)], vg_v)
        pltpu.sync_copy(sg_hbm.at[pl.ds(sub * RPT + s * 16, 16)], sg_v)
        pltpu.sync_copy(se_hbm.at[core, pl.ds(sub * RPT + s * 16, 16)], se_v)

        def batch(b, c2):
            pltpu.async_copy(y_hbm.at[vg_v.at[b]], rows_v, sem).wait()
            pltpu.async_copy(ic_hbm.at[sg_v.at[b]], scale_v, sem).wait()

            def mul_row(r, c3):
                for j in range(EMB // 16):
                    rows_v[r, pl.ds(j * 16, 16)] = (
                        rows_v[r, pl.ds(j * 16, 16)]
                        * scale_v[r, pl.ds(j * 16, 16)])
                return c3
            lax.fori_loop(0, EB, mul_row, 0)

            pltpu.sync_copy(rows_v, acc_sh.at[se_v.at[b]], add=True)
            return c2
        lax.fori_loop(0, 16, batch, 0)
        return c
    lax.fori_loop(0, SLABS, slab, 0)

    plsc.subcore_barrier()

    @pl.when(sub < NS - 1)
    def _():
        pltpu.sync_copy(acc_sh.at[pl.ds(sub * ETR, ETR)],
                        agg_hbm.at[core, pl.ds(sub * ETR, ETR)])

    @pl.when(sub == NS - 1)
    def _():
        pltpu.sync_copy(acc_sh.at[pl.ds((NS - 1) * ETR, DHALF - (NS - 1) * ETR)],
                        agg_hbm.at[core, pl.ds((NS - 1) * ETR, DHALF - (NS - 1) * ETR)])


_BN = 1000


def _tc_fold_body(p_ref, g_ref, out_ref):
    out_ref[...] = jnp.dot(p_ref[...], g_ref[...],
                           preferred_element_type=jnp.float32)


def _tc_fold(p_flat, g):
    return pl.pallas_call(
        _tc_fold_body,
        grid=(N // _BN,),
        in_specs=[
            pl.BlockSpec((_BN * 16, L), lambda i: (i, 0)),
            pl.BlockSpec((L, 8), lambda i: (0, 0)),
        ],
        out_specs=pl.BlockSpec((_BN * 16, 8), lambda i: (i, 0)),
        out_shape=jax.ShapeDtypeStruct((N * 16, 8), jnp.float32),
    )(p_flat, g)


def _tc_inv_body(cnt_ref, out_ref):
    out_ref[...] = 1.0 / jnp.maximum(cnt_ref[0], 1.0)


def _tc_inv(cnt10):
    return pl.pallas_call(
        _tc_inv_body,
        grid=(NREL, 10),
        in_specs=[pl.BlockSpec((1, 1, CR, EMB), lambda r, g: (g, r, 0, 0))],
        out_specs=pl.BlockSpec((1, CR, EMB), lambda r, g: (r, g, 0)),
        out_shape=jax.ShapeDtypeStruct((NREL, NPD, EMB), jnp.float32),
    )(cnt10)


def _tc_dense1_body(fse_ref, seg_ref, st_ref, w1_ref, b1_ref, w2_ref, b2_ref,
                    cw_ref, x0_ref, y_ref):
    f_se = fse_ref[...]
    seg = seg_ref[...]
    st = st_ref[...]
    st2 = st * st
    c0 = jnp.sum((seg == 0).astype(jnp.float32), axis=1, keepdims=True)
    c1 = jnp.sum((seg == 1).astype(jnp.float32), axis=1, keepdims=True)
    c2 = jnp.sum((seg == 2).astype(jnp.float32), axis=1, keepdims=True)
    q = c0 * st2[0:1, :] + c1 * st2[1:2, :] + c2 * st2[2:3, :]
    inv = 1.0 / jnp.maximum(jnp.sqrt(q), 1e-12)
    d0 = jnp.sum(inv * st[0:1, :], axis=1, keepdims=True)
    d1 = jnp.sum(inv * st[1:2, :], axis=1, keepdims=True)
    d2 = jnp.sum(inv * st[2:3, :], axis=1, keepdims=True)
    f_ge = jnp.where(seg == 0, d0, jnp.where(seg == 1, d1, d2))
    f = f_se + f_ge
    h = jnp.maximum(jnp.dot(f, w1_ref[...], preferred_element_type=jnp.float32)
                    + b1_ref[...], 0.0)
    x0 = jnp.dot(h, w2_ref[...], preferred_element_type=jnp.float32) + b2_ref[...]
    x0_ref[...] = x0
    for r in range(NREL):
        y_ref[r] = jnp.dot(x0, cw_ref[r], preferred_element_type=jnp.float32)


def _tc_dense1(f_se, seg, seg_tab_p, q_W1, q_b1, q_W2, q_b2, c1_w):
    return pl.pallas_call(
        _tc_dense1_body,
        grid=(N // _BN,),
        in_specs=[
            pl.BlockSpec((_BN, L), lambda i: (i, 0)),
            pl.BlockSpec((_BN, L), lambda i: (i, 0)),
            pl.BlockSpec((8, EMB), lambda i: (0, 0)),
            pl.BlockSpec((EMB, TXT), lambda i: (0, 0)),
            pl.BlockSpec((1, TXT), lambda i: (0, 0)),
            pl.BlockSpec((TXT, EMB), lambda i: (0, 0)),
            pl.BlockSpec((1, EMB), lambda i: (0, 0)),
            pl.BlockSpec((NREL, EMB, EMB), lambda i: (0, 0, 0)),
        ],
        out_specs=[
            pl.BlockSpec((_BN, EMB), lambda i: (i, 0)),
            pl.BlockSpec((NREL, _BN, EMB), lambda i: (0, i, 0)),
        ],
        out_shape=[
            jax.ShapeDtypeStruct((N, EMB), jnp.float32),
            jax.ShapeDtypeStruct((NREL, N, EMB), jnp.float32),
        ],
    )(f_se, seg, seg_tab_p, q_W1, q_b1, q_W2, q_b2, c1_w)


def _tc_dense2_body(x_ref, agg_ref, root_ref, b_ref, cw_ref, x1_ref, y_ref):
    x1 = jnp.maximum(
        jnp.dot(x_ref[...], root_ref[...], preferred_element_type=jnp.float32)
        + b_ref[...] + agg_ref[...], 0.0)
    x1_ref[...] = x1
    for r in range(NREL):
        y_ref[r] = jnp.dot(x1, cw_ref[r], preferred_element_type=jnp.float32)


def _tc_dense2(x, agg, root, b, c2_w):
    return pl.pallas_call(
        _tc_dense2_body,
        grid=(N // _BN,),
        in_specs=[
            pl.BlockSpec((_BN, EMB), lambda i: (i, 0)),
            pl.BlockSpec((_BN, EMB), lambda i: (i, 0)),
            pl.BlockSpec((EMB, EMB), lambda i: (0, 0)),
            pl.BlockSpec((1, EMB), lambda i: (0, 0)),
            pl.BlockSpec((NREL, EMB, EMB), lambda i: (0, 0, 0)),
        ],
        out_specs=[
            pl.BlockSpec((_BN, EMB), lambda i: (i, 0)),
            pl.BlockSpec((NREL, _BN, EMB), lambda i: (0, i, 0)),
        ],
        out_shape=[
            jax.ShapeDtypeStruct((N, EMB), jnp.float32),
            jax.ShapeDtypeStruct((NREL, N, EMB), jnp.float32),
        ],
    )(x, agg, root, b, c2_w)


def _tc_dense3_body(x_ref, agg_ref, root_ref, b_ref, lw_ref, lb_ref, out_ref):
    x2 = jnp.maximum(
        jnp.dot(x_ref[...], root_ref[...], preferred_element_type=jnp.float32)
        + b_ref[...] + agg_ref[...], 0.0)
    out_ref[...] = (jnp.dot(x2, lw_ref[...], preferred_element_type=jnp.float32)
                    + lb_ref[...])


def _tc_dense3(x, agg, root, b, lin_W, lin_b):
    return pl.pallas_call(
        _tc_dense3_body,
        grid=(N // _BN,),
        in_specs=[
            pl.BlockSpec((_BN, EMB), lambda i: (i, 0)),
            pl.BlockSpec((_BN, EMB), lambda i: (i, 0)),
            pl.BlockSpec((EMB, EMB), lambda i: (0, 0)),
            pl.BlockSpec((1, EMB), lambda i: (0, 0)),
            pl.BlockSpec((EMB, EMB), lambda i: (0, 0)),
            pl.BlockSpec((1, EMB), lambda i: (0, 0)),
        ],
        out_specs=pl.BlockSpec((_BN, EMB), lambda i: (i, 0)),
        out_shape=jax.ShapeDtypeStruct((N, EMB), jnp.float32),
    )(x, agg, root, b, lin_W, lin_b)


def kernel(src_tab, seg_tab, w, q_W1, q_b1, q_W2, q_b2,
           c1_w, c1_root, c1_b, c2_w, c2_root, c2_b, lin_W, lin_b,
           src, seg, edge_index, edge_type):
    src = src.astype(jnp.int32)
    seg = seg.astype(jnp.int32)
    esrc = edge_index[0].astype(jnp.int32)
    edst = edge_index[1].astype(jnp.int32)
    etype = edge_type.astype(jnp.int32)

    padi = jnp.zeros((EPAD - E,), jnp.int32)
    vgidx = jnp.concatenate([etype * N + esrc, padi]).reshape(EROWS, EB)
    sgidx = jnp.concatenate([etype * NPD + edst, padi]).reshape(EROWS, EB)
    se_l = [jnp.concatenate(
        [jnp.where((edst >= h * DHALF) & (edst < (h + 1) * DHALF),
                   edst - h * DHALF, EDUMMY),
         jnp.full((EPAD - E,), EDUMMY, jnp.int32)]) for h in range(NC)]
    seidx = jnp.stack(se_l).reshape(NC, EROWS, EB)
    sc_l = [jnp.concatenate(
        [jnp.where((edst >= g * CR) & (edst < (g + 1) * CR),
                   etype * CR + edst - g * CR, CDUMMY),
         jnp.full((EPAD - E,), CDUMMY, jnp.int32)]) for g in range(10)]
    scidx = jnp.stack(sc_l).reshape(10, EROWS, EB)

    seg_tab_p = jnp.pad(seg_tab, ((0, 5), (0, 0)))
    b1 = q_b1.reshape(1, TXT)
    b2 = q_b2.reshape(1, EMB)
    c1_b_r = c1_b.reshape(1, EMB)
    c2_b_r = c2_b.reshape(1, EMB)
    lin_b_r = lin_b.reshape(1, EMB)

    p_out = _sc_embed(src_tab, src.reshape(N * L), w)
    g = (jnp.arange(L, dtype=jnp.int32)[:, None] // 16
         == jnp.arange(8, dtype=jnp.int32)[None, :]).astype(jnp.float32)
    f_se = _tc_fold(p_out.reshape(N * 16, L), g).reshape(N, L)

    cnt10 = _sc_count(scidx)
    ic = _tc_inv(cnt10.reshape(10, NREL, CR, EMB)).reshape(NREL * NPD, EMB)

    x0, y0 = _tc_dense1(f_se, seg, seg_tab_p, q_W1, b1, q_W2, b2, c1_w)
    agg1 = _sc_edge(y0.reshape(NREL * N, EMB), ic, vgidx, sgidx, seidx)
    x1, y1 = _tc_dense2(x0, agg1.reshape(N, EMB), c1_root, c1_b_r, c2_w)
    agg2 = _sc_edge(y1.reshape(NREL * N, EMB), ic, vgidx, sgidx, seidx)
    return _tc_dense3(x1, agg2.reshape(N, EMB), c2_root, c2_b_r, lin_W, lin_b_r)
